# Initial kernel scaffold; baseline (speedup 1.0000x reference)
import jax
import jax.numpy as jnp
from jax.experimental import pallas as pl


def kernel(nodes_flat, adj_flat, batch, lengths, emb, ge_W1, ge_b1, ge_W2, ge_b2, ge_Wl, ge_bl, p_Wr, p_Wl, p_b, gf_W1, gf_b1, gf_W2, gf_b2, gf_Wl, gf_bl, lf_W, lf_b):
    raise NotImplementedError("write your pallas kernel here")


# speedup vs baseline: 117.6745x; 117.6745x over previous
import functools

import jax
import jax.numpy as jnp
from jax import lax
from jax.experimental import pallas as pl
from jax.experimental.pallas import tpu as pltpu
from jax.experimental.pallas import tpu_sc as plsc

_N = 512
_B = 16
_ES = 128
_WD = 300
_WDP = 384
_RATIO = 0.2
_HI = lax.Precision.HIGHEST

_NW = 32
_ROWS_PER_W = _N // _NW


def _sc_gather_body(table_hbm, idx_hbm, out_hbm, idx_v, rows_v, sem):
    wid = lax.axis_index("s") * 2 + lax.axis_index("c")
    base = wid * _ROWS_PER_W
    pltpu.sync_copy(idx_hbm.at[pl.ds(base, _ROWS_PER_W)], idx_v)
    pltpu.async_copy(table_hbm.at[idx_v], rows_v, sem).wait()
    pltpu.sync_copy(rows_v, out_hbm.at[pl.ds(base, _ROWS_PER_W)])


def _sc_gather(emb_padded, idx):
    k = pl.kernel(
        _sc_gather_body,
        out_type=jax.ShapeDtypeStruct((_N, _WDP), jnp.float32),
        mesh=plsc.VectorSubcoreMesh(core_axis_name="c", subcore_axis_name="s"),
        scratch_types=[
            pltpu.VMEM((_ROWS_PER_W,), jnp.int32),
            pltpu.VMEM((_ROWS_PER_W, _WDP), jnp.float32),
            pltpu.SemaphoreType.DMA,
        ],
    )
    return k(emb_padded, idx)


def _dense_body(adj_ref, x0_ref, brow_ref, bcol_ref,
                w1_ref, b1_ref, w2_ref, b2_ref, wl_ref, bl_ref,
                pwr_ref, pwl_ref, pb_ref,
                f1_ref, fb1_ref, f2_ref, fb2_ref, fl_ref, fbl_ref,
                lfw_ref, lfb_ref, out_ref):
    f32 = jnp.float32
    adj = adj_ref[...]
    w = ((adj + adj.T) > 0).astype(f32)
    deg = jnp.sum(w, axis=1, keepdims=True) + 1.0
    dinv = lax.rsqrt(deg)
    dinv2 = dinv * dinv

    def gcn(x, wt, b, dv, dv2, keep):
        h = jnp.dot(x, wt, precision=_HI)
        v = h * dv
        if keep is not None:
            v = v * keep
        u = jnp.dot(w, v, precision=_HI)
        if keep is not None:
            u = u * keep
        return u * dv + h * dv2 + b

    relu = lambda t: jnp.maximum(t, 0.0)

    x0 = x0_ref[...]
    x1 = relu(gcn(x0, w1_ref[...], b1_ref[...], dinv, dinv2, None))
    x2 = relu(gcn(x1, w2_ref[...], b2_ref[...], dinv, dinv2, None))
    x = relu(jnp.dot(jnp.concatenate([x1, x2], axis=1), wl_ref[...],
                     precision=_HI) + bl_ref[...])

    brow = brow_ref[...]
    bcol = bcol_ref[...]
    m_bn = (lax.broadcasted_iota(jnp.int32, (_B, _N), 0) == brow).astype(f32)
    m_nb = (lax.broadcasted_iota(jnp.int32, (_N, _B), 1) == bcol).astype(f32)
    counts = jnp.sum(m_bn, axis=1, keepdims=True)
    xs0 = jnp.dot(m_bn, x, precision=_HI) / jnp.maximum(counts, 1.0)

    aggr = jnp.dot(w, x, precision=_HI)
    score = (jnp.dot(aggr, pwl_ref[...], precision=_HI)
             + jnp.dot(x, pwr_ref[...], precision=_HI) + pb_ref[...])
    score_row = score.T

    kk = jnp.ceil(_RATIO * counts)
    k_node = jnp.dot(m_nb, kk, precision=_HI)

    same = bcol == brow
    ii = lax.broadcasted_iota(jnp.int32, (_N, _N), 0)
    jj = lax.broadcasted_iota(jnp.int32, (_N, _N), 1)
    beats = (score_row > score) | ((score_row == score) & (jj < ii))
    rank = jnp.sum(jnp.where(same & beats, 1.0, 0.0), axis=1, keepdims=True)
    keep = (rank < k_node).astype(f32)

    xg = x * jnp.tanh(score)
    deg2 = 1.0 + keep * jnp.dot(w, keep, precision=_HI)
    db = lax.rsqrt(deg2)
    db2 = db * db

    y1 = relu(gcn(xg, f1_ref[...], fb1_ref[...], db, db2, keep))
    y2 = relu(gcn(y1, f2_ref[...], fb2_ref[...], db, db2, keep))
    out2 = jnp.dot(jnp.concatenate([y1, y2], axis=1), fl_ref[...],
                   precision=_HI) + fbl_ref[...]

    c1 = jnp.dot(m_bn, keep, precision=_HI)
    xs1 = jnp.dot(m_bn, out2 * keep, precision=_HI) / jnp.maximum(c1, 1.0)

    feat = jnp.dot(jnp.concatenate([xs0, xs1], axis=1), lfw_ref[...],
                   precision=_HI) + lfb_ref[...]
    nrm = jnp.sqrt(jnp.sum(feat * feat, axis=1, keepdims=True))
    out_ref[...] = feat / (nrm + 1e-10)


_DENSE_CALL = pl.pallas_call(
    _dense_body,
    out_shape=jax.ShapeDtypeStruct((_B, _ES), jnp.float32),
)


def kernel(nodes_flat, adj_flat, batch, lengths, emb, ge_W1, ge_b1, ge_W2,
           ge_b2, ge_Wl, ge_bl, p_Wr, p_Wl, p_b, gf_W1, gf_b1, gf_W2, gf_b2,
           gf_Wl, gf_bl, lf_W, lf_b):
    del lengths
    emb_padded = jnp.pad(emb, ((0, 0), (0, _WDP - _WD)))
    w1_padded = jnp.pad(ge_W1, ((0, _WDP - _WD), (0, 0)))
    x0 = _sc_gather(emb_padded, nodes_flat.astype(jnp.int32))
    return _DENSE_CALL(
        adj_flat, x0,
        batch.astype(jnp.int32).reshape(1, _N),
        batch.astype(jnp.int32).reshape(_N, 1),
        w1_padded, ge_b1.reshape(1, _ES), ge_W2, ge_b2.reshape(1, _ES),
        ge_Wl, ge_bl.reshape(1, _ES),
        p_Wr, p_Wl, p_b.reshape(1, 1),
        gf_W1, gf_b1.reshape(1, _ES), gf_W2, gf_b2.reshape(1, _ES),
        gf_Wl, gf_bl.reshape(1, _ES),
        lf_W, lf_b.reshape(1, _ES))


# speedup vs baseline: 346.4023x; 2.9437x over previous
import jax
import jax.numpy as jnp
from jax import lax
from jax.experimental import pallas as pl
from jax.experimental.pallas import tpu as pltpu
from jax.experimental.pallas import tpu_sc as plsc

_N = 512
_B = 16
_ES = 128
_WD = 300
_RATIO = 0.2
_HI = lax.Precision.HIGHEST

_NW = 32
_RW = _N // _NW
_TAIL = _WD - 256


def _sc_gather_body(table, idx_hbm, o0, o1, o2, idx_v, r0, r1, r2, s0, s1, s2):
    wid = lax.axis_index("s") * 2 + lax.axis_index("c")
    base = wid * _RW
    pltpu.sync_copy(idx_hbm.at[pl.ds(base, _RW)], idx_v)
    c0 = pltpu.async_copy(table.at[idx_v, pl.ds(0, 128)], r0, s0)
    c1 = pltpu.async_copy(table.at[idx_v, pl.ds(128, 128)], r1, s1)
    iv = idx_v[...]
    tail = [pltpu.async_copy(table.at[pl.ds(iv[i], 1), pl.ds(256, _TAIL)],
                             r2.at[pl.ds(i, 1)], s2)
            for i in range(_RW)]
    c0.wait()
    c1.wait()
    for c in tail:
        c.wait()
    pltpu.sync_copy(r0, o0.at[pl.ds(base, _RW)])
    pltpu.sync_copy(r1, o1.at[pl.ds(base, _RW)])
    pltpu.sync_copy(r2, o2.at[pl.ds(base, _RW)])


def _sc_gather(emb, idx):
    k = pl.kernel(
        _sc_gather_body,
        out_type=[jax.ShapeDtypeStruct((_N, 128), jnp.float32),
                  jax.ShapeDtypeStruct((_N, 128), jnp.float32),
                  jax.ShapeDtypeStruct((_N, _TAIL), jnp.float32)],
        mesh=plsc.VectorSubcoreMesh(core_axis_name="c", subcore_axis_name="s"),
        scratch_types=[
            pltpu.VMEM((_RW,), jnp.int32),
            pltpu.VMEM((_RW, 128), jnp.float32),
            pltpu.VMEM((_RW, 128), jnp.float32),
            pltpu.VMEM((_RW, _TAIL), jnp.float32),
            pltpu.SemaphoreType.DMA,
            pltpu.SemaphoreType.DMA,
            pltpu.SemaphoreType.DMA,
        ],
    )
    return k(emb, idx)


def _dense_body(adj_ref, x0a_ref, x0b_ref, x0c_ref, brow_ref, bcol_ref,
                w1a_ref, w1b_ref, w1c_ref, b1_ref, w2_ref, b2_ref,
                wl_ref, bl_ref, pwr_ref, pwl_ref, pb_ref,
                f1_ref, fb1_ref, f2_ref, fb2_ref, fl_ref, fbl_ref,
                lfw_ref, lfb_ref, out_ref):
    f32 = jnp.float32
    adj = adj_ref[...]
    w = ((adj + adj.T) > 0).astype(f32)
    deg = jnp.sum(w, axis=1, keepdims=True) + 1.0
    dinv = lax.rsqrt(deg)
    dinv2 = dinv * dinv

    def gcn_h(h, b, dv, dv2, keep):
        v = h * dv
        if keep is not None:
            v = v * keep
        u = jnp.dot(w, v, precision=_HI)
        if keep is not None:
            u = u * keep
        return u * dv + h * dv2 + b

    relu = lambda t: jnp.maximum(t, 0.0)

    h1 = (jnp.dot(x0a_ref[...], w1a_ref[...], precision=_HI)
          + jnp.dot(x0b_ref[...], w1b_ref[...], precision=_HI)
          + jnp.dot(x0c_ref[...], w1c_ref[...], precision=_HI))
    x1 = relu(gcn_h(h1, b1_ref[...], dinv, dinv2, None))
    h2 = jnp.dot(x1, w2_ref[...], precision=_HI)
    x2 = relu(gcn_h(h2, b2_ref[...], dinv, dinv2, None))
    x = relu(jnp.dot(jnp.concatenate([x1, x2], axis=1), wl_ref[...],
                     precision=_HI) + bl_ref[...])

    brow = brow_ref[...]
    bcol = bcol_ref[...]
    m_bn = (lax.broadcasted_iota(jnp.int32, (_B, _N), 0) == brow).astype(f32)
    m_nb = (lax.broadcasted_iota(jnp.int32, (_N, _B), 1) == bcol).astype(f32)
    counts = jnp.sum(m_bn, axis=1, keepdims=True)
    xs0 = jnp.dot(m_bn, x, precision=_HI) / jnp.maximum(counts, 1.0)

    aggr = jnp.dot(w, x, precision=_HI)
    score = (jnp.dot(aggr, pwl_ref[...], precision=_HI)
             + jnp.dot(x, pwr_ref[...], precision=_HI) + pb_ref[...])
    score_row = score.T

    kk = jnp.ceil(_RATIO * counts)
    k_node = jnp.dot(m_nb, kk, precision=_HI)

    same = bcol == brow
    ii = lax.broadcasted_iota(jnp.int32, (_N, _N), 0)
    jj = lax.broadcasted_iota(jnp.int32, (_N, _N), 1)
    beats = (score_row > score) | ((score_row == score) & (jj < ii))
    rank = jnp.sum(jnp.where(same & beats, 1.0, 0.0), axis=1, keepdims=True)
    keep = (rank < k_node).astype(f32)

    xg = x * jnp.tanh(score)
    deg2 = 1.0 + keep * jnp.dot(w, keep, precision=_HI)
    db = lax.rsqrt(deg2)
    db2 = db * db

    g1 = jnp.dot(xg, f1_ref[...], precision=_HI)
    y1 = relu(gcn_h(g1, fb1_ref[...], db, db2, keep))
    g2 = jnp.dot(y1, f2_ref[...], precision=_HI)
    y2 = relu(gcn_h(g2, fb2_ref[...], db, db2, keep))
    out2 = jnp.dot(jnp.concatenate([y1, y2], axis=1), fl_ref[...],
                   precision=_HI) + fbl_ref[...]

    c1 = jnp.dot(m_bn, keep, precision=_HI)
    xs1 = jnp.dot(m_bn, out2 * keep, precision=_HI) / jnp.maximum(c1, 1.0)

    feat = jnp.dot(jnp.concatenate([xs0, xs1], axis=1), lfw_ref[...],
                   precision=_HI) + lfb_ref[...]
    nrm = jnp.sqrt(jnp.sum(feat * feat, axis=1, keepdims=True))
    out_ref[...] = feat / (nrm + 1e-10)


_DENSE_CALL = pl.pallas_call(
    _dense_body,
    out_shape=jax.ShapeDtypeStruct((_B, _ES), jnp.float32),
)


def kernel(nodes_flat, adj_flat, batch, lengths, emb, ge_W1, ge_b1, ge_W2,
           ge_b2, ge_Wl, ge_bl, p_Wr, p_Wl, p_b, gf_W1, gf_b1, gf_W2, gf_b2,
           gf_Wl, gf_bl, lf_W, lf_b):
    del lengths
    x0a, x0b, x0c = _sc_gather(emb, nodes_flat.astype(jnp.int32))
    return _DENSE_CALL(
        adj_flat, x0a, x0b, x0c,
        batch.astype(jnp.int32).reshape(1, _N),
        batch.astype(jnp.int32).reshape(_N, 1),
        ge_W1[:128], ge_W1[128:256], ge_W1[256:],
        ge_b1.reshape(1, _ES), ge_W2, ge_b2.reshape(1, _ES),
        ge_Wl, ge_bl.reshape(1, _ES),
        p_Wr, p_Wl, p_b.reshape(1, 1),
        gf_W1, gf_b1.reshape(1, _ES), gf_W2, gf_b2.reshape(1, _ES),
        gf_Wl, gf_bl.reshape(1, _ES),
        lf_W, lf_b.reshape(1, _ES))


# speedup vs baseline: 470.7504x; 1.3590x over previous
import jax
import jax.numpy as jnp
from jax import lax
from jax.experimental import pallas as pl
from jax.experimental.pallas import tpu as pltpu
from jax.experimental.pallas import tpu_sc as plsc

_N = 512
_B = 16
_ES = 128
_WD = 300
_V = 30000
_RATIO = 0.2
_HI = lax.Precision.HIGHEST
_VCHUNK = 2048

def _proj_body(embt_ref, w1_ref, out_ref):
    out_ref[...] = lax.dot_general(
        embt_ref[...], w1_ref[...],
        dimension_numbers=(((0,), (0,)), ((), ())))


_PROJ_CALL = pl.pallas_call(
    _proj_body,
    grid=(_V // _VCHUNK + (_V % _VCHUNK != 0),),
    in_specs=[
        pl.BlockSpec((_WD, _VCHUNK), lambda j: (0, j)),
        pl.BlockSpec((_WD, _ES), lambda j: (0, 0)),
    ],
    out_specs=pl.BlockSpec((_VCHUNK, _ES), lambda j: (j, 0)),
    out_shape=jax.ShapeDtypeStruct((_V, _ES), jnp.float32),
)

_NW = 32
_RW = _N // _NW


def _sc_gather_body(table, idx_hbm, out, idx_v, rows_v, sem):
    wid = lax.axis_index("s") * 2 + lax.axis_index("c")
    base = wid * _RW
    pltpu.sync_copy(idx_hbm.at[pl.ds(base, _RW)], idx_v)
    pltpu.async_copy(table.at[idx_v], rows_v, sem).wait()
    pltpu.sync_copy(rows_v, out.at[pl.ds(base, _RW)])


def _sc_gather(table, idx):
    k = pl.kernel(
        _sc_gather_body,
        out_type=jax.ShapeDtypeStruct((_N, _ES), jnp.float32),
        mesh=plsc.VectorSubcoreMesh(core_axis_name="c", subcore_axis_name="s"),
        scratch_types=[
            pltpu.VMEM((_RW,), jnp.int32),
            pltpu.VMEM((_RW, _ES), jnp.float32),
            pltpu.SemaphoreType.DMA,
        ],
    )
    return k(table, idx)


def _dense_body(adj_ref, h1_ref, brow_ref,
                b1_ref, w2_ref, b2_ref, wl_ref, bl_ref,
                pwr_ref, pwl_ref, pb_ref,
                f1_ref, fb1_ref, f2_ref, fb2_ref, fl_ref, fbl_ref,
                lfw_ref, lfb_ref, out_ref):
    f32 = jnp.float32
    adj = adj_ref[...]
    w = ((adj + adj.T) > 0).astype(f32)
    deg = jnp.sum(w, axis=1, keepdims=True) + 1.0
    dinv = lax.rsqrt(deg)
    dinv2 = dinv * dinv

    def gcn_h(h, b, dv, dv2, keep):
        v = h * dv
        if keep is not None:
            v = v * keep
        u = jnp.dot(w, v, precision=_HI)
        if keep is not None:
            u = u * keep
        return u * dv + h * dv2 + b

    relu = lambda t: jnp.maximum(t, 0.0)

    x1 = relu(gcn_h(h1_ref[...], b1_ref[...], dinv, dinv2, None))
    h2 = jnp.dot(x1, w2_ref[...])
    x2 = relu(gcn_h(h2, b2_ref[...], dinv, dinv2, None))
    x = relu(jnp.dot(jnp.concatenate([x1, x2], axis=1), wl_ref[...])
             + bl_ref[...])

    brow = brow_ref[...]
    bcol = brow.T
    m_bn = (lax.broadcasted_iota(jnp.int32, (_B, _N), 0) == brow).astype(f32)
    m_nb = (lax.broadcasted_iota(jnp.int32, (_N, _B), 1) == bcol).astype(f32)
    counts = jnp.sum(m_bn, axis=1, keepdims=True)
    xs0 = jnp.dot(m_bn, x, precision=_HI) / jnp.maximum(counts, 1.0)

    aggr = jnp.dot(w, x, precision=_HI)
    score = (jnp.dot(aggr, pwl_ref[...])
             + jnp.dot(x, pwr_ref[...]) + pb_ref[...])
    score_row = score.T

    kk = jnp.ceil(_RATIO * counts)
    k_node = jnp.dot(m_nb, kk, precision=_HI)

    same = bcol == brow
    ii = lax.broadcasted_iota(jnp.int32, (_N, _N), 0)
    jj = lax.broadcasted_iota(jnp.int32, (_N, _N), 1)
    beats = (score_row > score) | ((score_row == score) & (jj < ii))
    rank = jnp.sum(jnp.where(same & beats, 1.0, 0.0), axis=1, keepdims=True)
    keep = (rank < k_node).astype(f32)

    xg = x * jnp.tanh(score)
    deg2 = 1.0 + keep * jnp.dot(w, keep, precision=_HI)
    db = lax.rsqrt(deg2)
    db2 = db * db

    g1 = jnp.dot(xg, f1_ref[...])
    y1 = relu(gcn_h(g1, fb1_ref[...], db, db2, keep))
    g2 = jnp.dot(y1, f2_ref[...])
    y2 = relu(gcn_h(g2, fb2_ref[...], db, db2, keep))
    out2 = jnp.dot(jnp.concatenate([y1, y2], axis=1), fl_ref[...]) + fbl_ref[...]

    c1 = jnp.dot(m_bn, keep, precision=_HI)
    xs1 = jnp.dot(m_bn, out2 * keep, precision=_HI) / jnp.maximum(c1, 1.0)

    feat = jnp.dot(jnp.concatenate([xs0, xs1], axis=1), lfw_ref[...]) + lfb_ref[...]
    nrm = jnp.sqrt(jnp.sum(feat * feat, axis=1, keepdims=True))
    out_ref[...] = feat / (nrm + 1e-10)


_DENSE_CALL = pl.pallas_call(
    _dense_body,
    out_shape=jax.ShapeDtypeStruct((_B, _ES), jnp.float32),
)


def kernel(nodes_flat, adj_flat, batch, lengths, emb, ge_W1, ge_b1, ge_W2,
           ge_b2, ge_Wl, ge_bl, p_Wr, p_Wl, p_b, gf_W1, gf_b1, gf_W2, gf_b2,
           gf_Wl, gf_bl, lf_W, lf_b):
    del lengths
    p = _PROJ_CALL(emb.T, ge_W1)
    h1 = _sc_gather(p, nodes_flat.astype(jnp.int32))
    return _DENSE_CALL(
        adj_flat, h1,
        batch.astype(jnp.int32).reshape(1, _N),
        ge_b1.reshape(1, _ES), ge_W2, ge_b2.reshape(1, _ES),
        ge_Wl, ge_bl.reshape(1, _ES),
        p_Wr, p_Wl, p_b.reshape(1, 1),
        gf_W1, gf_b1.reshape(1, _ES), gf_W2, gf_b2.reshape(1, _ES),
        gf_Wl, gf_bl.reshape(1, _ES),
        lf_W, lf_b.reshape(1, _ES))


# speedup vs baseline: 517.3787x; 1.0991x over previous
import jax
import jax.numpy as jnp
from jax import lax
from jax.experimental import pallas as pl
from jax.experimental.pallas import tpu as pltpu
from jax.experimental.pallas import tpu_sc as plsc

_N = 512
_B = 16
_ES = 128
_WD = 300
_V = 30000
_RATIO = 0.2
_HI = lax.Precision.HIGHEST
_VCHUNK = 7680

def _proj_body(embt_ref, w1_ref, out_ref):
    out_ref[...] = lax.dot_general(
        embt_ref[...], w1_ref[...],
        dimension_numbers=(((0,), (0,)), ((), ())))


_PROJ_CALL = pl.pallas_call(
    _proj_body,
    grid=(_V // _VCHUNK + (_V % _VCHUNK != 0),),
    in_specs=[
        pl.BlockSpec((_WD, _VCHUNK), lambda j: (0, j)),
        pl.BlockSpec((_WD, _ES), lambda j: (0, 0)),
    ],
    out_specs=pl.BlockSpec((_VCHUNK, _ES), lambda j: (j, 0)),
    out_shape=jax.ShapeDtypeStruct((_V, _ES), jnp.float32),
)

_NW = 32
_RW = _N // _NW


def _sc_gather_body(table, idx_hbm, out, idx_v, rows_v, sem):
    wid = lax.axis_index("s") * 2 + lax.axis_index("c")
    base = wid * _RW
    pltpu.sync_copy(idx_hbm.at[pl.ds(base, _RW)], idx_v)
    pltpu.async_copy(table.at[idx_v], rows_v, sem).wait()
    pltpu.sync_copy(rows_v, out.at[pl.ds(base, _RW)])


def _sc_gather(table, idx):
    k = pl.kernel(
        _sc_gather_body,
        out_type=jax.ShapeDtypeStruct((_N, _ES), jnp.float32),
        mesh=plsc.VectorSubcoreMesh(core_axis_name="c", subcore_axis_name="s"),
        scratch_types=[
            pltpu.VMEM((_RW,), jnp.int32),
            pltpu.VMEM((_RW, _ES), jnp.float32),
            pltpu.SemaphoreType.DMA,
        ],
    )
    return k(table, idx)


def _dense_body(adj_ref, h1_ref, brow_ref,
                b1_ref, w2_ref, b2_ref, wl_ref, bl_ref,
                pwr_ref, pwl_ref, pb_ref,
                f1_ref, fb1_ref, f2_ref, fb2_ref, fl_ref, fbl_ref,
                lfw_ref, lfb_ref, out_ref):
    f32 = jnp.float32
    adj = adj_ref[...]
    w = ((adj + adj.T) > 0).astype(f32)
    deg = jnp.sum(w, axis=1, keepdims=True) + 1.0
    dinv = lax.rsqrt(deg)
    dinv2 = dinv * dinv

    def gcn_h(h, b, dv, dv2, keep):
        v = h * dv
        if keep is not None:
            v = v * keep
        u = jnp.dot(w, v, precision=_HI)
        if keep is not None:
            u = u * keep
        return u * dv + h * dv2 + b

    relu = lambda t: jnp.maximum(t, 0.0)

    x1 = relu(gcn_h(h1_ref[...], b1_ref[...], dinv, dinv2, None))
    h2 = jnp.dot(x1, w2_ref[...])
    x2 = relu(gcn_h(h2, b2_ref[...], dinv, dinv2, None))
    x = relu(jnp.dot(jnp.concatenate([x1, x2], axis=1), wl_ref[...])
             + bl_ref[...])

    brow = brow_ref[...]
    bcol = brow.T
    m_bn = (lax.broadcasted_iota(jnp.int32, (_B, _N), 0) == brow).astype(f32)
    m_nb = (lax.broadcasted_iota(jnp.int32, (_N, _B), 1) == bcol).astype(f32)
    counts = jnp.sum(m_bn, axis=1, keepdims=True)
    xs0 = jnp.dot(m_bn, x, precision=_HI) / jnp.maximum(counts, 1.0)

    aggr = jnp.dot(w, x, precision=_HI)
    score = (jnp.dot(aggr, pwl_ref[...])
             + jnp.dot(x, pwr_ref[...]) + pb_ref[...])
    score_row = score.T

    kk = jnp.ceil(_RATIO * counts)
    k_node = jnp.dot(m_nb, kk, precision=_HI)

    same = bcol == brow
    ii = lax.broadcasted_iota(jnp.int32, (_N, _N), 0)
    jj = lax.broadcasted_iota(jnp.int32, (_N, _N), 1)
    beats = (score_row > score) | ((score_row == score) & (jj < ii))
    rank = jnp.sum(jnp.where(same & beats, 1.0, 0.0), axis=1, keepdims=True)
    keep = (rank < k_node).astype(f32)

    xg = x * jnp.tanh(score)
    deg2 = 1.0 + keep * jnp.dot(w, keep, precision=_HI)
    db = lax.rsqrt(deg2)
    db2 = db * db

    g1 = jnp.dot(xg, f1_ref[...])
    y1 = relu(gcn_h(g1, fb1_ref[...], db, db2, keep))
    g2 = jnp.dot(y1, f2_ref[...])
    y2 = relu(gcn_h(g2, fb2_ref[...], db, db2, keep))
    out2 = jnp.dot(jnp.concatenate([y1, y2], axis=1), fl_ref[...]) + fbl_ref[...]

    c1 = jnp.dot(m_bn, keep, precision=_HI)
    xs1 = jnp.dot(m_bn, out2 * keep, precision=_HI) / jnp.maximum(c1, 1.0)

    feat = jnp.dot(jnp.concatenate([xs0, xs1], axis=1), lfw_ref[...]) + lfb_ref[...]
    nrm = jnp.sqrt(jnp.sum(feat * feat, axis=1, keepdims=True))
    out_ref[...] = feat / (nrm + 1e-10)


_DENSE_CALL = pl.pallas_call(
    _dense_body,
    out_shape=jax.ShapeDtypeStruct((_B, _ES), jnp.float32),
)


def kernel(nodes_flat, adj_flat, batch, lengths, emb, ge_W1, ge_b1, ge_W2,
           ge_b2, ge_Wl, ge_bl, p_Wr, p_Wl, p_b, gf_W1, gf_b1, gf_W2, gf_b2,
           gf_Wl, gf_bl, lf_W, lf_b):
    del lengths
    p = _PROJ_CALL(emb.T, ge_W1)
    h1 = _sc_gather(p, nodes_flat.astype(jnp.int32))
    return _DENSE_CALL(
        adj_flat, h1,
        batch.astype(jnp.int32).reshape(1, _N),
        ge_b1.reshape(1, _ES), ge_W2, ge_b2.reshape(1, _ES),
        ge_Wl, ge_bl.reshape(1, _ES),
        p_Wr, p_Wl, p_b.reshape(1, 1),
        gf_W1, gf_b1.reshape(1, _ES), gf_W2, gf_b2.reshape(1, _ES),
        gf_Wl, gf_bl.reshape(1, _ES),
        lf_W, lf_b.reshape(1, _ES))
